# baseline (speedup 1.0000x reference)
import functools

import jax
import jax.numpy as jnp
from jax import lax
from jax.experimental import pallas as pl
from jax.experimental.pallas import tpu as pltpu
from jax.experimental.pallas import tpu_sc as plsc

D = 256
K = 8192
KBLK = 512
NKB = K // KBLK
TBLK = 512
_CROSS_PREC = lax.Precision.DEFAULT


def _argmin_body(r_ref, w_ref, idx_ref, mn_ref, mi_ref):
    kb = pl.program_id(0)
    tb = pl.program_id(1)
    rows = pl.ds(tb * TBLK, TBLK)
    rs = r_ref[...]
    wb = w_ref[...]
    cross = lax.dot_general(
        rs, wb, (((1,), (1,)), ((), ())),
        preferred_element_type=jnp.float32,
        precision=_CROSS_PREC,
    )
    wsq = wb * wb
    w2r = lax.dot_general(
        jnp.ones((1, D), jnp.float32), wsq, (((1,), (1,)), ((), ())),
        preferred_element_type=jnp.float32,
        precision=lax.Precision.HIGHEST,
    )
    d2 = w2r - 2.0 * cross
    bm = jnp.min(d2, axis=1, keepdims=True)
    iot = lax.broadcasted_iota(jnp.int32, d2.shape, 1).astype(jnp.float32)
    bif = jnp.min(jnp.where(d2 == bm, iot, float(KBLK)), axis=1, keepdims=True)
    bi = bif.astype(jnp.int32) + kb * KBLK

    @pl.when(kb == 0)
    def _():
        mn_ref[rows, :] = bm
        mi_ref[rows, :] = bi

    @pl.when(kb > 0)
    def _():
        upd = bm < mn_ref[rows, :]
        mi_ref[rows, :] = jnp.where(upd, bi, mi_ref[rows, :])
        mn_ref[rows, :] = jnp.where(upd, bm, mn_ref[rows, :])

    idx_ref[...] = mi_ref[rows, :]


def _argmin_level(r, Wl):
    n = r.shape[0]
    ntb = n // TBLK
    idx = pl.pallas_call(
        _argmin_body,
        grid=(NKB, ntb),
        in_specs=[
            pl.BlockSpec((TBLK, D), lambda kb, tb: (tb, 0)),
            pl.BlockSpec((KBLK, D), lambda kb, tb: (kb, 0)),
        ],
        out_specs=pl.BlockSpec((TBLK, 1), lambda kb, tb: (tb, 0)),
        out_shape=jax.ShapeDtypeStruct((n, 1), jnp.int32),
        scratch_shapes=[
            pltpu.VMEM((n, 1), jnp.float32),
            pltpu.VMEM((n, 1), jnp.int32),
        ],
    )(r, Wl)
    return idx.reshape(n)


@functools.cache
def _make_sc_gather(n):
    info = plsc.get_sparse_core_info()
    nw = info.num_cores * info.num_subcores
    b_per_w = n // nw
    nch = -(-b_per_w // 72)
    ch = b_per_w // nch
    assert ch * nch == b_per_w and ch % 8 == 0
    mesh = plsc.VectorSubcoreMesh(core_axis_name="c", subcore_axis_name="s")

    @functools.partial(
        pl.kernel, mesh=mesh,
        out_type=jax.ShapeDtypeStruct((n, D), jnp.float32),
        scratch_types=[
            pltpu.VMEM((nch, ch), jnp.int32),
            pltpu.VMEM((nch, ch, D), jnp.float32),
            pltpu.SemaphoreType.DMA,
        ],
    )
    def gather_k(table_hbm, idx_hbm, out_hbm, idx_v, rows_v, sem):
        wid = lax.axis_index("s") * info.num_cores + lax.axis_index("c")
        base = wid * b_per_w
        for j in range(nch):
            pltpu.sync_copy(idx_hbm.at[pl.ds(base + j * ch, ch)], idx_v.at[j])
        copies = [
            pltpu.async_copy(table_hbm.at[idx_v.at[j]], rows_v.at[j], sem)
            for j in range(nch)
        ]
        for c in copies:
            c.wait()
        for j in range(nch):
            pltpu.sync_copy(rows_v.at[j], out_hbm.at[pl.ds(base + j * ch, ch)])

    return gather_k


def kernel(x, W):
    b, t, d = x.shape
    n = b * t
    x2 = x.reshape(n, d)
    r = x2
    gather = _make_sc_gather(n)
    idxs = []
    for l in range(W.shape[0]):
        idx = _argmin_level(r, W[l])
        q = gather(W[l], idx)
        r = r - q
        idxs.append(idx)
    qsum = x2 - r
    quant = x2 + lax.stop_gradient(qsum - x2)
    return (quant.reshape(b, t, d),
            jnp.stack(idxs, axis=-1).reshape(b, t, len(idxs)))


# speedup vs baseline: 1.0238x; 1.0238x over previous
import functools

import jax
import jax.numpy as jnp
from jax import lax
from jax.experimental import pallas as pl
from jax.experimental.pallas import tpu as pltpu
from jax.experimental.pallas import tpu_sc as plsc

D = 256
K = 8192
KBLK = 512
NKB = K // KBLK
TBLK = 512
_CROSS_PREC = lax.Precision.DEFAULT


def _argmin_body(r_ref, q_ref, w_ref, idx_ref, rn_ref, rs_ref, mn_ref, mi_ref):
    kb = pl.program_id(1)

    @pl.when(kb == 0)
    def _():
        rs_ref[...] = r_ref[...] - q_ref[...]
        rn_ref[...] = rs_ref[...]

    rs2 = rs_ref[...] + rs_ref[...]
    wb = w_ref[pl.ds(kb * KBLK, KBLK), :]
    cross2 = lax.dot_general(
        rs2, wb, (((1,), (1,)), ((), ())),
        preferred_element_type=jnp.float32,
        precision=_CROSS_PREC,
    )
    wsq = wb * wb
    w2r = lax.dot_general(
        jnp.ones((1, D), jnp.float32), wsq, (((1,), (1,)), ((), ())),
        preferred_element_type=jnp.float32,
        precision=lax.Precision.HIGHEST,
    )
    d2 = w2r - cross2
    bm = jnp.min(d2, axis=1, keepdims=True)
    iot = lax.broadcasted_iota(jnp.int32, d2.shape, 1).astype(jnp.float32)
    bif = jnp.min(jnp.where(d2 == bm, iot, float(KBLK)), axis=1, keepdims=True)
    bi = bif.astype(jnp.int32) + kb * KBLK

    @pl.when(kb == 0)
    def _():
        mn_ref[...] = bm
        mi_ref[...] = bi

    @pl.when(kb > 0)
    def _():
        upd = bm < mn_ref[...]
        mi_ref[...] = jnp.where(upd, bi, mi_ref[...])
        mn_ref[...] = jnp.where(upd, bm, mn_ref[...])

    @pl.when(kb == NKB - 1)
    def _():
        idx_ref[...] = mi_ref[...]


def _argmin_level(r, qprev, Wl):
    n = r.shape[0]
    ntb = n // TBLK
    idx, rnew = pl.pallas_call(
        _argmin_body,
        grid=(ntb, NKB),
        in_specs=[
            pl.BlockSpec((TBLK, D), lambda tb, kb: (tb, 0)),
            pl.BlockSpec((TBLK, D), lambda tb, kb: (tb, 0)),
            pl.BlockSpec((K, D), lambda tb, kb: (0, 0)),
        ],
        out_specs=[
            pl.BlockSpec((TBLK, 1), lambda tb, kb: (tb, 0)),
            pl.BlockSpec((TBLK, D), lambda tb, kb: (tb, 0)),
        ],
        out_shape=[
            jax.ShapeDtypeStruct((n, 1), jnp.int32),
            jax.ShapeDtypeStruct((n, D), jnp.float32),
        ],
        scratch_shapes=[
            pltpu.VMEM((TBLK, D), jnp.float32),
            pltpu.VMEM((TBLK, 1), jnp.float32),
            pltpu.VMEM((TBLK, 1), jnp.int32),
        ],
        compiler_params=pltpu.CompilerParams(
            dimension_semantics=("parallel", "arbitrary"),
        ),
    )(r, qprev, Wl)
    return idx.reshape(n), rnew


@functools.cache
def _make_sc_gather(n):
    info = plsc.get_sparse_core_info()
    nw = info.num_cores * info.num_subcores
    b_per_w = n // nw
    nch = -(-b_per_w // 72)
    ch = b_per_w // nch
    assert ch * nch == b_per_w and ch % 8 == 0
    mesh = plsc.VectorSubcoreMesh(core_axis_name="c", subcore_axis_name="s")

    @functools.partial(
        pl.kernel, mesh=mesh,
        out_type=jax.ShapeDtypeStruct((n, D), jnp.float32),
        scratch_types=[
            pltpu.VMEM((nch, ch), jnp.int32),
            pltpu.VMEM((nch, ch, D), jnp.float32),
            pltpu.SemaphoreType.DMA,
        ],
    )
    def gather_k(table_hbm, idx_hbm, out_hbm, idx_v, rows_v, sem):
        wid = lax.axis_index("s") * info.num_cores + lax.axis_index("c")
        base = wid * b_per_w
        for j in range(nch):
            pltpu.sync_copy(idx_hbm.at[pl.ds(base + j * ch, ch)], idx_v.at[j])
        copies = [
            pltpu.async_copy(table_hbm.at[idx_v.at[j]], rows_v.at[j], sem)
            for j in range(nch)
        ]
        for c in copies:
            c.wait()
        for j in range(nch):
            pltpu.sync_copy(rows_v.at[j], out_hbm.at[pl.ds(base + j * ch, ch)])

    return gather_k


def kernel(x, W):
    b, t, d = x.shape
    n = b * t
    x2 = x.reshape(n, d)
    r = x2
    q = jnp.zeros_like(x2)
    gather = _make_sc_gather(n)
    idxs = []
    for l in range(W.shape[0]):
        idx, r = _argmin_level(r, q, W[l])
        q = gather(W[l], idx)
        idxs.append(idx)
    qsum = x2 - (r - q)
    quant = x2 + lax.stop_gradient(qsum - x2)
    return (quant.reshape(b, t, d),
            jnp.stack(idxs, axis=-1).reshape(b, t, len(idxs)))


# speedup vs baseline: 1.7357x; 1.6954x over previous
import functools

import jax
import jax.numpy as jnp
from jax import lax
from jax.experimental import pallas as pl
from jax.experimental.pallas import tpu as pltpu
from jax.experimental.pallas import tpu_sc as plsc

D = 256
K = 8192
KBLK = 512
NKB = K // KBLK
TBLK = 512
_CROSS_PREC = lax.Precision.DEFAULT


def _argmin_body(r_ref, q_ref, w_ref, idx_ref, rn_ref, rs_ref, mn_ref, mi_ref):
    kb = pl.program_id(1)

    @pl.when(kb == 0)
    def _():
        rs_ref[...] = r_ref[...] - q_ref[...]
        rn_ref[...] = rs_ref[...]

    rs2 = rs_ref[...] + rs_ref[...]
    wb = w_ref[pl.ds(kb * KBLK, KBLK), :]
    cross2 = lax.dot_general(
        wb, rs2, (((1,), (1,)), ((), ())),
        preferred_element_type=jnp.float32,
        precision=_CROSS_PREC,
    )
    wsq = wb * wb
    w2c = lax.dot_general(
        wsq, jnp.ones((1, D), jnp.float32), (((1,), (1,)), ((), ())),
        preferred_element_type=jnp.float32,
        precision=lax.Precision.HIGHEST,
    )
    d2 = w2c - cross2
    bm = jnp.min(d2, axis=0, keepdims=True)
    iot = lax.broadcasted_iota(jnp.int32, d2.shape, 0).astype(jnp.float32)
    bif = jnp.min(jnp.where(d2 == bm, iot, float(KBLK)), axis=0, keepdims=True)
    bi = bif.astype(jnp.int32) + kb * KBLK

    @pl.when(kb == 0)
    def _():
        mn_ref[...] = bm
        mi_ref[...] = bi

    @pl.when(kb > 0)
    def _():
        upd = bm < mn_ref[...]
        mi_ref[...] = jnp.where(upd, bi, mi_ref[...])
        mn_ref[...] = jnp.where(upd, bm, mn_ref[...])

    @pl.when(kb == NKB - 1)
    def _():
        idx_ref[...] = mi_ref[...].reshape(1, 1, TBLK)


def _argmin_level(r, qprev, Wl):
    n = r.shape[0]
    ntb = n // TBLK
    idx, rnew = pl.pallas_call(
        _argmin_body,
        grid=(ntb, NKB),
        in_specs=[
            pl.BlockSpec((TBLK, D), lambda tb, kb: (tb, 0)),
            pl.BlockSpec((TBLK, D), lambda tb, kb: (tb, 0)),
            pl.BlockSpec((K, D), lambda tb, kb: (0, 0)),
        ],
        out_specs=[
            pl.BlockSpec((1, 1, TBLK), lambda tb, kb: (tb, 0, 0)),
            pl.BlockSpec((TBLK, D), lambda tb, kb: (tb, 0)),
        ],
        out_shape=[
            jax.ShapeDtypeStruct((ntb, 1, TBLK), jnp.int32),
            jax.ShapeDtypeStruct((n, D), jnp.float32),
        ],
        scratch_shapes=[
            pltpu.VMEM((TBLK, D), jnp.float32),
            pltpu.VMEM((1, TBLK), jnp.float32),
            pltpu.VMEM((1, TBLK), jnp.int32),
        ],
        compiler_params=pltpu.CompilerParams(
            dimension_semantics=("parallel", "arbitrary"),
        ),
    )(r, qprev, Wl)
    return idx.reshape(n), rnew


@functools.cache
def _make_sc_gather(n):
    info = plsc.get_sparse_core_info()
    nw = info.num_cores * info.num_subcores
    b_per_w = n // nw
    nch = -(-b_per_w // 72)
    ch = b_per_w // nch
    assert ch * nch == b_per_w and ch % 8 == 0
    mesh = plsc.VectorSubcoreMesh(core_axis_name="c", subcore_axis_name="s")

    @functools.partial(
        pl.kernel, mesh=mesh,
        out_type=jax.ShapeDtypeStruct((n, D), jnp.float32),
        scratch_types=[
            pltpu.VMEM((nch, ch), jnp.int32),
            pltpu.VMEM((nch, ch, D), jnp.float32),
            pltpu.SemaphoreType.DMA,
        ],
    )
    def gather_k(table_hbm, idx_hbm, out_hbm, idx_v, rows_v, sem):
        wid = lax.axis_index("s") * info.num_cores + lax.axis_index("c")
        base = wid * b_per_w
        for j in range(nch):
            pltpu.sync_copy(idx_hbm.at[pl.ds(base + j * ch, ch)], idx_v.at[j])
        copies = [
            pltpu.async_copy(table_hbm.at[idx_v.at[j]], rows_v.at[j], sem)
            for j in range(nch)
        ]
        for c in copies:
            c.wait()
        for j in range(nch):
            pltpu.sync_copy(rows_v.at[j], out_hbm.at[pl.ds(base + j * ch, ch)])

    return gather_k


def kernel(x, W):
    b, t, d = x.shape
    n = b * t
    x2 = x.reshape(n, d)
    r = x2
    q = jnp.zeros_like(x2)
    gather = _make_sc_gather(n)
    idxs = []
    for l in range(W.shape[0]):
        idx, r = _argmin_level(r, q, W[l])
        q = gather(W[l], idx)
        idxs.append(idx)
    qsum = x2 - (r - q)
    quant = x2 + lax.stop_gradient(qsum - x2)
    return (quant.reshape(b, t, d),
            jnp.stack(idxs, axis=-1).reshape(b, t, len(idxs)))
